# baseline (speedup 1.0000x reference)
import jax
import jax.numpy as jnp
from jax import lax
from jax.experimental import pallas as pl
from jax.experimental.pallas import tpu as pltpu
from jax.experimental.pallas import tpu_sc as plsc

N_NODE = 100000
D = 128
B_POS = 16384
B_NEG = 81920

NC = 2
NS = 16
NW = NC * NS
LANES = 16
CHUNK = 128

POS_BLKS = B_POS // CHUNK
NEG_BLKS = B_NEG // CHUNK
POS_PER_W = POS_BLKS // NW
NEG_PER_W = NEG_BLKS // NW
DSUB = D // LANES


GROUPS = CHUNK // LANES


def _sc_body(pf, pt, nf, nt, e1, e2, e2c, o1, o2, on,
             ia, ib, ra, rb, p1, gsem):
    wid = lax.axis_index("s") * NC + lax.axis_index("c")

    def dots(buf):
        lidx = lax.iota(jnp.int32, LANES)

        @plsc.parallel_loop(0, GROUPS, 1, unroll=1)
        def _(g):
            tot = jnp.zeros((LANES,), jnp.float32)
            for j in range(LANES):
                r = g * LANES + j
                acc = ra[buf, r, pl.ds(0, LANES)] * rb[buf, r, pl.ds(0, LANES)]
                for b in range(1, DSUB):
                    acc = acc + (ra[buf, r, pl.ds(LANES * b, LANES)]
                                 * rb[buf, r, pl.ds(LANES * b, LANES)])
                tot = jnp.where(lidx == j, jnp.sum(acc), tot)
            p1[pl.ds(g * LANES, LANES)] = tot

    def phase(idx_f_hbm, idx_t_hbm, tab_a, tab_b, out_hbm, nchunks):
        base = wid * nchunks
        pltpu.sync_copy(idx_f_hbm.at[wid], ia.at[pl.ds(0, nchunks)])
        pltpu.sync_copy(idx_t_hbm.at[wid], ib.at[pl.ds(0, nchunks)])

        def fetch(c, buf):
            pltpu.async_copy(tab_a.at[ia.at[c]], ra.at[buf], gsem.at[buf])
            pltpu.async_copy(tab_b.at[ib.at[c]], rb.at[buf], gsem.at[buf])

        def consume(c, buf):
            pltpu.make_async_copy(tab_a.at[ia.at[c]], ra.at[buf],
                                  gsem.at[buf]).wait()
            pltpu.make_async_copy(tab_b.at[ib.at[c]], rb.at[buf],
                                  gsem.at[buf]).wait()
            dots(buf)
            pltpu.sync_copy(p1, out_hbm.at[base + c])

        fetch(0, 0)

        def step(i, carry):
            for b in range(2):
                cc = i * 2 + b

                @pl.when(cc + 1 < nchunks)
                def _():
                    fetch(cc + 1, 1 - b)

                consume(cc, b)
            return carry

        lax.fori_loop(0, nchunks // 2, step, 0)

    phase(pf, pt, e1, e1, o1, POS_PER_W)
    phase(pf, pt, e2, e2c, o2, POS_PER_W)
    phase(nf, nt, e2, e2c, on, NEG_PER_W)


_sc_dots = pl.kernel(
    _sc_body,
    out_type=(
        jax.ShapeDtypeStruct((POS_BLKS, CHUNK), jnp.float32),
        jax.ShapeDtypeStruct((POS_BLKS, CHUNK), jnp.float32),
        jax.ShapeDtypeStruct((NEG_BLKS, CHUNK), jnp.float32),
    ),
    mesh=plsc.VectorSubcoreMesh(core_axis_name="c", subcore_axis_name="s"),
    compiler_params=pltpu.CompilerParams(needs_layout_passes=False),
    scratch_types=(
        pltpu.VMEM((NEG_PER_W, CHUNK), jnp.int32),
        pltpu.VMEM((NEG_PER_W, CHUNK), jnp.int32),
        pltpu.VMEM((2, CHUNK, D), jnp.float32),
        pltpu.VMEM((2, CHUNK, D), jnp.float32),
        pltpu.VMEM((CHUNK,), jnp.float32),
        pltpu.SemaphoreType.DMA((2,)),
    ),
)


def _log_sigmoid(x):
    return jnp.minimum(x, 0.0) - jnp.log(1.0 + jnp.exp(-jnp.abs(x)))


def _reduce_body(o1, o2, on, w, first_ref, second_ref):
    first = -jnp.sum(w[...] * _log_sigmoid(o1[...]))
    pos_loss = jnp.sum(_log_sigmoid(o2[...]))
    neg_loss = jnp.sum(_log_sigmoid(-on[...]))
    first_ref[0, 0] = first
    second_ref[0, 0] = -(pos_loss + neg_loss)


_reduce = pl.pallas_call(
    _reduce_body,
    out_shape=(
        jax.ShapeDtypeStruct((1, 1), jnp.float32),
        jax.ShapeDtypeStruct((1, 1), jnp.float32),
    ),
    out_specs=(
        pl.BlockSpec(memory_space=pltpu.SMEM),
        pl.BlockSpec(memory_space=pltpu.SMEM),
    ),
)


def kernel(pos, pos_w, neg, embed_1, embed_2, embed_2_context):
    pf = pos[:, 0].reshape(NW, POS_PER_W, CHUNK)
    pt = pos[:, 1].reshape(NW, POS_PER_W, CHUNK)
    nf = neg[:, 0].reshape(NW, NEG_PER_W, CHUNK)
    nt = neg[:, 1].reshape(NW, NEG_PER_W, CHUNK)

    o1, o2, on = _sc_dots(pf, pt, nf, nt, embed_1, embed_2, embed_2_context)

    first, second = _reduce(o1, o2, on, pos_w.reshape(POS_BLKS, CHUNK))
    return first[0, 0], second[0, 0]


# speedup vs baseline: 1.6992x; 1.6992x over previous
import jax
import jax.numpy as jnp
from jax import lax
from jax.experimental import pallas as pl
from jax.experimental.pallas import tpu as pltpu
from jax.experimental.pallas import tpu_sc as plsc

N_NODE = 100000
D = 128
B_POS = 16384
B_NEG = 81920

NC = 2
NS = 16
NW = NC * NS
LANES = 16
CHUNK = 128

POS_BLKS = B_POS // CHUNK
NEG_BLKS = B_NEG // CHUNK
POS_PER_W = POS_BLKS // NW
NEG_PER_W = NEG_BLKS // NW
DSUB = D // LANES


GROUPS = CHUNK // LANES


def _sc_body(pf, pt, nf, nt, e1, e2, e2c, o1, o2, on,
             ia, ib, ra, rb, tbuf, p1, gsem):
    wid = lax.axis_index("s") * NC + lax.axis_index("c")

    def dots(buf):
        lane16 = lax.iota(jnp.int32, LANES) * LANES

        @plsc.parallel_loop(0, GROUPS, 1, unroll=1)
        def _(g):
            tb = g * (LANES * LANES)
            for j in range(LANES):
                r = g * LANES + j
                acc = ra[buf, r, pl.ds(0, LANES)] * rb[buf, r, pl.ds(0, LANES)]
                for b in range(1, DSUB):
                    acc = acc + (ra[buf, r, pl.ds(LANES * b, LANES)]
                                 * rb[buf, r, pl.ds(LANES * b, LANES)])
                plsc.store_scatter(tbuf, [lane16 + (tb + j)], acc)
            tot = tbuf[pl.ds(tb, LANES)]
            for l in range(1, LANES):
                tot = tot + tbuf[pl.ds(tb + l * LANES, LANES)]
            p1[pl.ds(g * LANES, LANES)] = tot

    def phase(idx_f_hbm, idx_t_hbm, tab_a, tab_b, out_hbm, nchunks):
        base = wid * nchunks
        pltpu.sync_copy(idx_f_hbm.at[wid], ia.at[pl.ds(0, nchunks)])
        pltpu.sync_copy(idx_t_hbm.at[wid], ib.at[pl.ds(0, nchunks)])

        def fetch(c, buf):
            pltpu.async_copy(tab_a.at[ia.at[c]], ra.at[buf], gsem.at[buf])
            pltpu.async_copy(tab_b.at[ib.at[c]], rb.at[buf], gsem.at[buf])

        def consume(c, buf):
            pltpu.make_async_copy(tab_a.at[ia.at[c]], ra.at[buf],
                                  gsem.at[buf]).wait()
            pltpu.make_async_copy(tab_b.at[ib.at[c]], rb.at[buf],
                                  gsem.at[buf]).wait()
            dots(buf)
            pltpu.sync_copy(p1, out_hbm.at[base + c])

        fetch(0, 0)

        def step(i, carry):
            for b in range(2):
                cc = i * 2 + b

                @pl.when(cc + 1 < nchunks)
                def _():
                    fetch(cc + 1, 1 - b)

                consume(cc, b)
            return carry

        lax.fori_loop(0, nchunks // 2, step, 0)

    phase(pf, pt, e1, e1, o1, POS_PER_W)
    phase(pf, pt, e2, e2c, o2, POS_PER_W)
    phase(nf, nt, e2, e2c, on, NEG_PER_W)


_sc_dots = pl.kernel(
    _sc_body,
    out_type=(
        jax.ShapeDtypeStruct((POS_BLKS, CHUNK), jnp.float32),
        jax.ShapeDtypeStruct((POS_BLKS, CHUNK), jnp.float32),
        jax.ShapeDtypeStruct((NEG_BLKS, CHUNK), jnp.float32),
    ),
    mesh=plsc.VectorSubcoreMesh(core_axis_name="c", subcore_axis_name="s"),
    compiler_params=pltpu.CompilerParams(needs_layout_passes=False),
    scratch_types=(
        pltpu.VMEM((NEG_PER_W, CHUNK), jnp.int32),
        pltpu.VMEM((NEG_PER_W, CHUNK), jnp.int32),
        pltpu.VMEM((2, CHUNK, D), jnp.float32),
        pltpu.VMEM((2, CHUNK, D), jnp.float32),
        pltpu.VMEM((CHUNK * LANES,), jnp.float32),
        pltpu.VMEM((CHUNK,), jnp.float32),
        pltpu.SemaphoreType.DMA((2,)),
    ),
)


def _log_sigmoid(x):
    return jnp.minimum(x, 0.0) - jnp.log(1.0 + jnp.exp(-jnp.abs(x)))


def _reduce_body(o1, o2, on, w, first_ref, second_ref):
    first = -jnp.sum(w[...] * _log_sigmoid(o1[...]))
    pos_loss = jnp.sum(_log_sigmoid(o2[...]))
    neg_loss = jnp.sum(_log_sigmoid(-on[...]))
    first_ref[0, 0] = first
    second_ref[0, 0] = -(pos_loss + neg_loss)


_reduce = pl.pallas_call(
    _reduce_body,
    out_shape=(
        jax.ShapeDtypeStruct((1, 1), jnp.float32),
        jax.ShapeDtypeStruct((1, 1), jnp.float32),
    ),
    out_specs=(
        pl.BlockSpec(memory_space=pltpu.SMEM),
        pl.BlockSpec(memory_space=pltpu.SMEM),
    ),
)


def kernel(pos, pos_w, neg, embed_1, embed_2, embed_2_context):
    pf = pos[:, 0].reshape(NW, POS_PER_W, CHUNK)
    pt = pos[:, 1].reshape(NW, POS_PER_W, CHUNK)
    nf = neg[:, 0].reshape(NW, NEG_PER_W, CHUNK)
    nt = neg[:, 1].reshape(NW, NEG_PER_W, CHUNK)

    o1, o2, on = _sc_dots(pf, pt, nf, nt, embed_1, embed_2, embed_2_context)

    first, second = _reduce(o1, o2, on, pos_w.reshape(POS_BLKS, CHUNK))
    return first[0, 0], second[0, 0]


# speedup vs baseline: 2.0559x; 1.2099x over previous
import jax
import jax.numpy as jnp
from jax import lax
from jax.experimental import pallas as pl
from jax.experimental.pallas import tpu as pltpu
from jax.experimental.pallas import tpu_sc as plsc

N_NODE = 100000
D = 128
B_POS = 16384
B_NEG = 81920

NC = 2
NS = 16
NW = NC * NS
LANES = 16
CHUNK = 128

POS_BLKS = B_POS // CHUNK
NEG_BLKS = B_NEG // CHUNK
POS_PER_W = POS_BLKS // NW
NEG_PER_W = NEG_BLKS // NW
DSUB = D // LANES


GROUPS = CHUNK // LANES
NBUF = 3


def _sc_body(pf, pt, nf, nt, e1, e2, e2c, o1, o2, on,
             ia, ib, ra, rb, tbuf, p1, gsem):
    wid = lax.axis_index("s") * NC + lax.axis_index("c")

    def dots(buf):
        lane16 = lax.iota(jnp.int32, LANES) * LANES

        @plsc.parallel_loop(0, GROUPS, 1, unroll=1)
        def _(g):
            tb = g * (LANES * LANES)
            for j in range(LANES):
                r = g * LANES + j
                acc = ra[buf, r, pl.ds(0, LANES)] * rb[buf, r, pl.ds(0, LANES)]
                for b in range(1, DSUB):
                    acc = acc + (ra[buf, r, pl.ds(LANES * b, LANES)]
                                 * rb[buf, r, pl.ds(LANES * b, LANES)])
                plsc.store_scatter(tbuf, [lane16 + (tb + j)], acc)
            tot = tbuf[pl.ds(tb, LANES)]
            for l in range(1, LANES):
                tot = tot + tbuf[pl.ds(tb + l * LANES, LANES)]
            p1[pl.ds(g * LANES, LANES)] = tot

    def phase(idx_f_hbm, idx_t_hbm, tab_a, tab_b, out_hbm, nchunks):
        base = wid * nchunks
        pltpu.sync_copy(idx_f_hbm.at[wid], ia.at[pl.ds(0, nchunks)])
        pltpu.sync_copy(idx_t_hbm.at[wid], ib.at[pl.ds(0, nchunks)])

        def fetch(c, buf):
            pltpu.async_copy(tab_a.at[ia.at[c]], ra.at[buf], gsem.at[buf])
            pltpu.async_copy(tab_b.at[ib.at[c]], rb.at[buf], gsem.at[buf])

        def consume(c, buf):
            pltpu.make_async_copy(tab_a.at[ia.at[c]], ra.at[buf],
                                  gsem.at[buf]).wait()
            pltpu.make_async_copy(tab_b.at[ib.at[c]], rb.at[buf],
                                  gsem.at[buf]).wait()
            dots(buf)
            pltpu.sync_copy(p1, out_hbm.at[base + c])

        fetch(0, 0)
        fetch(1, 1)

        def step(cc, carry):
            nxt = cc + (NBUF - 1)

            @pl.when(nxt < nchunks)
            def _():
                fetch(nxt, lax.rem(nxt, NBUF))

            consume(cc, lax.rem(cc, NBUF))
            return carry

        lax.fori_loop(0, nchunks, step, 0)

    phase(pf, pt, e1, e1, o1, POS_PER_W)
    phase(pf, pt, e2, e2c, o2, POS_PER_W)
    phase(nf, nt, e2, e2c, on, NEG_PER_W)


_sc_dots = pl.kernel(
    _sc_body,
    out_type=(
        jax.ShapeDtypeStruct((POS_BLKS, CHUNK), jnp.float32),
        jax.ShapeDtypeStruct((POS_BLKS, CHUNK), jnp.float32),
        jax.ShapeDtypeStruct((NEG_BLKS, CHUNK), jnp.float32),
    ),
    mesh=plsc.VectorSubcoreMesh(core_axis_name="c", subcore_axis_name="s"),
    compiler_params=pltpu.CompilerParams(needs_layout_passes=False),
    scratch_types=(
        pltpu.VMEM((NEG_PER_W, CHUNK), jnp.int32),
        pltpu.VMEM((NEG_PER_W, CHUNK), jnp.int32),
        pltpu.VMEM((NBUF, CHUNK, D), jnp.float32),
        pltpu.VMEM((NBUF, CHUNK, D), jnp.float32),
        pltpu.VMEM((CHUNK * LANES,), jnp.float32),
        pltpu.VMEM((CHUNK,), jnp.float32),
        pltpu.SemaphoreType.DMA((NBUF,)),
    ),
)


def _log_sigmoid(x):
    return jnp.minimum(x, 0.0) - jnp.log(1.0 + jnp.exp(-jnp.abs(x)))


def _reduce_body(o1, o2, on, w, first_ref, second_ref):
    first = -jnp.sum(w[...] * _log_sigmoid(o1[...]))
    pos_loss = jnp.sum(_log_sigmoid(o2[...]))
    neg_loss = jnp.sum(_log_sigmoid(-on[...]))
    first_ref[0, 0] = first
    second_ref[0, 0] = -(pos_loss + neg_loss)


_reduce = pl.pallas_call(
    _reduce_body,
    out_shape=(
        jax.ShapeDtypeStruct((1, 1), jnp.float32),
        jax.ShapeDtypeStruct((1, 1), jnp.float32),
    ),
    out_specs=(
        pl.BlockSpec(memory_space=pltpu.SMEM),
        pl.BlockSpec(memory_space=pltpu.SMEM),
    ),
)


def kernel(pos, pos_w, neg, embed_1, embed_2, embed_2_context):
    pf = pos[:, 0].reshape(NW, POS_PER_W, CHUNK)
    pt = pos[:, 1].reshape(NW, POS_PER_W, CHUNK)
    nf = neg[:, 0].reshape(NW, NEG_PER_W, CHUNK)
    nt = neg[:, 1].reshape(NW, NEG_PER_W, CHUNK)

    o1, o2, on = _sc_dots(pf, pt, nf, nt, embed_1, embed_2, embed_2_context)

    first, second = _reduce(o1, o2, on, pos_w.reshape(POS_BLKS, CHUNK))
    return first[0, 0], second[0, 0]
